# baseline (speedup 1.0000x reference)
import functools

import jax
import jax.numpy as jnp
from jax import lax
from jax.experimental import pallas as pl
from jax.experimental.pallas import tpu as pltpu
from jax.experimental.pallas import tpu_sc as plsc

B, T, N, C, K = 8, 100, 196, 256, 16
ROWS = B * K
NCORES, NSUB = 2, 16
NW = NCORES * NSUB
RPW = ROWS // NW
WPB = K // RPW

_mesh = plsc.VectorSubcoreMesh(
    core_axis_name="c", subcore_axis_name="s",
    num_cores=NCORES, num_subcores=NSUB)


@functools.partial(
    pl.kernel,
    out_type=(
        jax.ShapeDtypeStruct((B, N, K, C), jnp.float32),
        jax.ShapeDtypeStruct((ROWS, C), jnp.float32),
    ),
    mesh=_mesh,
    scratch_types=[
        pltpu.VMEM((B, K), jnp.int32),
        pltpu.VMEM((N, C // 2), jnp.float32),
        pltpu.VMEM((N, C // 2), jnp.float32),
        pltpu.VMEM((N, C // 2), jnp.float32),
        pltpu.VMEM((N, C // 2), jnp.float32),
        pltpu.VMEM((RPW, C), jnp.float32),
        pltpu.SemaphoreType.DMA,
        pltpu.SemaphoreType.DMA,
        pltpu.SemaphoreType.DMA,
        pltpu.SemaphoreType.DMA,
        pltpu.SemaphoreType.DMA,
        pltpu.SemaphoreType.DMA,
        pltpu.SemaphoreType.DMA,
        pltpu.SemaphoreType.DMA,
        pltpu.SemaphoreType.DMA,
    ],
)
def _sc_gather(tks_hbm, patch_hbm, audio_hbm, outp_hbm, outa_hbm,
               kidx, buf0, buf1, buf2, buf3, abuf,
               sg0, sg1, sg2, sg3, sw0, sw1, sw2, sw3, sa):
    wid = lax.axis_index("s") * NCORES + lax.axis_index("c")
    base = wid * RPW
    b = wid // WPB
    k0 = (wid % WPB) * RPW

    pltpu.sync_copy(tks_hbm, kidx)
    row = kidx[b]
    lanes = lax.iota(jnp.int32, 16)
    sel = jnp.minimum(lanes + jax.lax.broadcast_in_dim(k0, (16,), ()),
                      jnp.full((16,), K - 1, jnp.int32))
    v16 = row.at[sel].get(mode="promise_in_bounds")

    NU = 2 * RPW
    bufs = (buf0, buf1, buf2, buf3)
    sgs = (sg0, sg1, sg2, sg3)
    sws = (sw0, sw1, sw2, sw3)

    def src_slice(u):
        r, h = u >> 1, u & 1
        return patch_hbm.at[v16[r], :, b, pl.ds(h * (C // 2), C // 2)]

    def dst_slice(u):
        r, h = u >> 1, u & 1
        return outp_hbm.at[b, :, k0 + r, pl.ds(h * (C // 2), C // 2)]

    gh = [None] * NU
    wh = [None] * NU
    for u in range(4):
        gh[u] = pltpu.async_copy(src_slice(u), bufs[u], sgs[u])

    ah = []
    for r in range(RPW):
        ah.append(pltpu.async_copy(
            audio_hbm.at[v16[r], b, :], abuf.at[r], sa))

    for u in range(NU):
        bsel = u % 4
        gh[u].wait()
        wh[u] = pltpu.async_copy(bufs[bsel], dst_slice(u), sws[bsel])
        if u + 4 < NU:
            wh[u].wait()
            gh[u + 4] = pltpu.async_copy(
                src_slice(u + 4), bufs[bsel], sgs[bsel])
    for u in range(NU - 4, NU):
        wh[u].wait()

    for h in ah:
        h.wait()
    pltpu.sync_copy(abuf, outa_hbm.at[pl.ds(base, RPW)])


def kernel(top_k_index_sort, patch_feat, audio_feat):
    outp, outa = _sc_gather(
        top_k_index_sort.reshape(B, K).astype(jnp.int32),
        jnp.transpose(patch_feat, (1, 2, 0, 3)),
        jnp.transpose(audio_feat, (1, 0, 2)),
    )
    return (jnp.transpose(outp, (0, 2, 1, 3)),
            outa.reshape(B, K, C))


# speedup vs baseline: 1.0034x; 1.0034x over previous
import functools

import jax
import jax.numpy as jnp
from jax import lax
from jax.experimental import pallas as pl
from jax.experimental.pallas import tpu as pltpu
from jax.experimental.pallas import tpu_sc as plsc

B, T, N, C, K = 8, 100, 196, 256, 16
ROWS = B * K
NCORES, NSUB = 2, 16
NW = NCORES * NSUB
RPW = ROWS // NW
WPB = K // RPW

_mesh = plsc.VectorSubcoreMesh(
    core_axis_name="c", subcore_axis_name="s",
    num_cores=NCORES, num_subcores=NSUB)


@functools.partial(
    pl.kernel,
    out_type=(
        jax.ShapeDtypeStruct((B, N, K, C), jnp.float32),
        jax.ShapeDtypeStruct((ROWS, C), jnp.float32),
    ),
    mesh=_mesh,
    scratch_types=[
        pltpu.VMEM((B, K), jnp.int32),
        pltpu.VMEM((N, C // 2), jnp.float32),
        pltpu.VMEM((N, C // 2), jnp.float32),
        pltpu.VMEM((N, C // 2), jnp.float32),
        pltpu.VMEM((N, C // 2), jnp.float32),
        pltpu.VMEM((RPW, C), jnp.float32),
        pltpu.SemaphoreType.DMA,
        pltpu.SemaphoreType.DMA,
        pltpu.SemaphoreType.DMA,
        pltpu.SemaphoreType.DMA,
        pltpu.SemaphoreType.DMA,
        pltpu.SemaphoreType.DMA,
        pltpu.SemaphoreType.DMA,
        pltpu.SemaphoreType.DMA,
        pltpu.SemaphoreType.DMA,
    ],
)
def _sc_gather(tks_hbm, patch_hbm, audio_hbm, outp_hbm, outa_hbm,
               kidx, buf0, buf1, buf2, buf3, abuf,
               sg0, sg1, sg2, sg3, sw0, sw1, sw2, sw3, sa):
    wid = lax.axis_index("s") * NCORES + lax.axis_index("c")
    base = wid * RPW
    b = wid // WPB
    k0 = (wid % WPB) * RPW

    pltpu.sync_copy(tks_hbm, kidx)
    row = kidx[b]
    lanes = lax.iota(jnp.int32, 16)
    sel = jnp.minimum(lanes + jax.lax.broadcast_in_dim(k0, (16,), ()),
                      jnp.full((16,), K - 1, jnp.int32))
    v16 = row.at[sel].get(mode="promise_in_bounds")

    ah = []
    for r in range(RPW):
        ah.append(pltpu.async_copy(
            audio_hbm.at[v16[r], b, :], abuf.at[r], sa))

    NU = 2 * RPW
    bufs = (buf0, buf1, buf2, buf3)
    sgs = (sg0, sg1, sg2, sg3)
    sws = (sw0, sw1, sw2, sw3)

    def src_slice(u):
        r, h = u >> 1, u & 1
        return patch_hbm.at[v16[r], :, b, pl.ds(h * (C // 2), C // 2)]

    def dst_slice(u):
        r, h = u >> 1, u & 1
        return outp_hbm.at[b, :, k0 + r, pl.ds(h * (C // 2), C // 2)]

    gh = [None] * NU
    wh = [None] * NU
    for u in range(4):
        gh[u] = pltpu.async_copy(src_slice(u), bufs[u], sgs[u])
    for u in range(NU):
        bsel = u % 4
        gh[u].wait()
        wh[u] = pltpu.async_copy(bufs[bsel], dst_slice(u), sws[bsel])
        if u + 4 < NU:
            wh[u].wait()
            gh[u + 4] = pltpu.async_copy(
                src_slice(u + 4), bufs[bsel], sgs[bsel])
    for u in range(NU - 4, NU):
        wh[u].wait()

    for h in ah:
        h.wait()
    pltpu.sync_copy(abuf, outa_hbm.at[pl.ds(base, RPW)])


def kernel(top_k_index_sort, patch_feat, audio_feat):
    outp, outa = _sc_gather(
        top_k_index_sort.reshape(B, K).astype(jnp.int32),
        jnp.transpose(patch_feat, (1, 2, 0, 3)),
        jnp.transpose(audio_feat, (1, 0, 2)),
    )
    return (jnp.transpose(outp, (0, 2, 1, 3)),
            outa.reshape(B, K, C))
